# baseline (speedup 1.0000x reference)
import functools

import jax
import jax.numpy as jnp
from jax import lax
from jax.experimental import pallas as pl
from jax.experimental.pallas import tpu as pltpu
from jax.experimental.pallas import tpu_sc as plsc

D = 2048
E = 8
F = 2048
S = 2048
K = 2
A = S * K
LBW = 0.01

BT = 256
NBR = S // BT
BLK = 128
NB = A // BLK
T = NB + E - 1


def _router_body(x_ref, gw_ref, gamma_ref, beta_ref,
                 xn_ref, meta_ref, counts_ref, aux_ref, run_ref, psum_ref):
    i = pl.program_id(0)

    @pl.when(i == 0)
    def _():
        run_ref[...] = jnp.zeros((1, E), jnp.float32)
        psum_ref[...] = jnp.zeros((1, E), jnp.float32)

    xb = x_ref[...]
    mean = jnp.mean(xb, axis=1, keepdims=True)
    xc = xb - mean
    var = jnp.mean(xc * xc, axis=1, keepdims=True)
    xn = xc / jnp.sqrt(var + 1e-5) * gamma_ref[...] + beta_ref[...]
    xn_ref[...] = xn

    logits = lax.dot_general(xn.astype(jnp.bfloat16),
                             gw_ref[...].astype(jnp.bfloat16),
                             (((1,), (1,)), ((), ())),
                             preferred_element_type=jnp.float32)

    ecol = lax.broadcasted_iota(jnp.int32, (BT, E), 1)
    m1 = jnp.max(logits, axis=1, keepdims=True)
    i1 = jnp.min(jnp.where(logits == m1, ecol, E), axis=1, keepdims=True)
    l2 = jnp.where(ecol == i1, jnp.float32(-1e30), logits)
    m2 = jnp.max(l2, axis=1, keepdims=True)
    i2 = jnp.min(jnp.where(l2 == m2, ecol, E), axis=1, keepdims=True)

    ed = jnp.exp(m2 - m1)
    g1 = 1.0 / (1.0 + ed)
    g2 = ed / (1.0 + ed)

    p = jnp.exp(logits - m1)
    p = p / jnp.sum(p, axis=1, keepdims=True)
    psum_ref[...] += jnp.sum(p, axis=0, keepdims=True)

    eh = jnp.concatenate([i1, i2], axis=0)
    onehot = (eh == lax.broadcasted_iota(jnp.int32, (2 * BT, E), 1)
              ).astype(jnp.float32)
    rl = lax.broadcasted_iota(jnp.int32, (2 * BT, 2 * BT), 0)
    cl = lax.broadcasted_iota(jnp.int32, (2 * BT, 2 * BT), 1)
    tril = (cl < rl).astype(jnp.float32)
    csum = lax.dot_general(tril, onehot, (((1,), (0,)), ((), ())),
                           preferred_element_type=jnp.float32)
    run = run_ref[...]
    rank = jnp.sum((csum + run) * onehot, axis=1, keepdims=True)
    run_ref[...] = run + jnp.sum(onehot, axis=0, keepdims=True)
    counts_ref[...] = run_ref[...]

    z = jnp.zeros((BT, 1), jnp.float32)
    meta_ref[...] = jnp.concatenate(
        [i1.astype(jnp.float32), i2.astype(jnp.float32),
         rank[:BT], rank[BT:], g1, g2, z, z], axis=1)[None]

    @pl.when(i == NBR - 1)
    def _():
        avg = psum_ref[...] / S
        mu = jnp.mean(avg)
        varu = jnp.sum((avg - mu) ** 2) / (E - 1)
        aux_ref[...] = jnp.full((1, E), varu / ((mu + 1e-6) ** 2) * LBW,
                                jnp.float32)


def _router_call(xm, gate_W, gamma, beta):
    return pl.pallas_call(
        _router_body,
        grid=(NBR,),
        in_specs=[
            pl.BlockSpec((BT, D), lambda i: (i, 0)),
            pl.BlockSpec((E, D), lambda i: (0, 0)),
            pl.BlockSpec((1, D), lambda i: (0, 0)),
            pl.BlockSpec((1, D), lambda i: (0, 0)),
        ],
        out_specs=[
            pl.BlockSpec((BT, D), lambda i: (i, 0)),
            pl.BlockSpec((1, BT, E), lambda i: (i, 0, 0)),
            pl.BlockSpec((1, E), lambda i: (0, 0)),
            pl.BlockSpec((1, E), lambda i: (0, 0)),
        ],
        out_shape=[
            jax.ShapeDtypeStruct((S, D), jnp.float32),
            jax.ShapeDtypeStruct((NBR, BT, E), jnp.float32),
            jax.ShapeDtypeStruct((1, E), jnp.float32),
            jax.ShapeDtypeStruct((1, E), jnp.float32),
        ],
        scratch_shapes=[
            pltpu.VMEM((1, E), jnp.float32),
            pltpu.VMEM((1, E), jnp.float32),
        ],
    )(xm, gate_W, gamma, beta)


def _dispatch_call(e_flat, r_flat, off16):
    mesh = plsc.VectorSubcoreMesh(core_axis_name="c", subcore_axis_name="s")

    @functools.partial(
        pl.kernel,
        out_type=(jax.ShapeDtypeStruct((A,), jnp.int32),
                  jax.ShapeDtypeStruct((A,), jnp.int32)),
        mesh=mesh,
        compiler_params=pltpu.CompilerParams(needs_layout_passes=False),
        scratch_types=[
            pltpu.VMEM((16,), jnp.int32),
            pltpu.VMEM((A,), jnp.int32),
            pltpu.VMEM((A,), jnp.int32),
            pltpu.VMEM((A,), jnp.int32),
            pltpu.VMEM((A,), jnp.int32),
        ],
    )
    def k(e_hbm, r_hbm, c_hbm, pos_hbm, stok_hbm, off_v, e_v, r_v, pos_v,
          stok_v):
        c = lax.axis_index("c")
        s = lax.axis_index("s")

        @pl.when((c == 0) & (s == 0))
        def _():
            pltpu.sync_copy(c_hbm, off_v)
            pltpu.sync_copy(e_hbm, e_v)
            pltpu.sync_copy(r_hbm, r_v)

            def body(i, carry):
                ev = e_v[pl.ds(i * 16, 16)]
                rv = r_v[pl.ds(i * 16, 16)]
                ps = rv + plsc.load_gather(off_v, [ev])
                pos_v[pl.ds(i * 16, 16)] = ps
                a = lax.iota(jnp.int32, 16) + i * 16
                plsc.store_scatter(stok_v, [ps], lax.rem(a, S))
                return carry

            lax.fori_loop(0, A // 16, body, 0)
            pltpu.sync_copy(pos_v, pos_hbm)
            pltpu.sync_copy(stok_v, stok_hbm)

    return k(e_flat, r_flat, off16)


def _i32view(a):
    n, d = a.shape
    return lax.bitcast_convert_type(a.reshape(n, d // 2, 2), jnp.int32)


def _bf16view(a):
    n, d2 = a.shape
    return lax.bitcast_convert_type(a, jnp.bfloat16).reshape(n, 2 * d2)


def _gather_rows_call(idx2, src):
    mesh = plsc.VectorSubcoreMesh(core_axis_name="c", subcore_axis_name="s")
    dt = src.dtype
    D2 = src.shape[1]

    @functools.partial(
        pl.kernel,
        out_type=jax.ShapeDtypeStruct((A, D2), dt),
        mesh=mesh,
        compiler_params=pltpu.CompilerParams(needs_layout_passes=False),
        scratch_types=[
            pltpu.VMEM((8, 16), jnp.int32),
            pltpu.VMEM((16, D2), dt),
            pltpu.VMEM((16, D2), dt),
            pltpu.SemaphoreType.DMA,
            pltpu.SemaphoreType.DMA,
        ],
    )
    def k(idx_hbm, src_hbm, out_hbm, idx_v, buf0, buf1, sem0, sem1):
        c = lax.axis_index("c")
        s = lax.axis_index("s")
        wid = s * 2 + c
        pltpu.sync_copy(idx_hbm.at[pl.ds(wid * 8, 8)], idx_v)
        bufs = (buf0, buf1)
        sems = (sem0, sem1)
        pend = {0: pltpu.async_copy(src_hbm.at[idx_v.at[0]], buf0, sem0)}
        for j in range(8):
            if j + 1 < 8:
                pend[(j + 1) % 2] = pltpu.async_copy(
                    src_hbm.at[idx_v.at[j + 1]], bufs[(j + 1) % 2],
                    sems[(j + 1) % 2])
            pend[j % 2].wait()
            pltpu.sync_copy(bufs[j % 2],
                            out_hbm.at[pl.ds(wid * 128 + j * 16, 16)])

    return k(idx2, src)


FC = 2
FH = F // FC
NSLOT = 3

TB_E, TB_B, TB_Z, TB_V, TB_WF, TB_CS0, TB_CS1, TB_PA, TB_PB = range(9)


def _ffn_body(tab_ref, soff_ref, x_ref, w1_any, w2_any, y_ref,
              w1s, w2s, sem1, sem2):
    t = pl.program_id(0)
    fc = pl.program_id(1)
    e = tab_ref[TB_E, t]
    b = tab_ref[TB_B, t]

    def start_pair_chunk(pe, pc, slot):
        pltpu.make_async_copy(w1_any.at[pe, pl.ds(pc * FH, FH), :],
                              w1s.at[slot], sem1.at[slot]).start()
        pltpu.make_async_copy(w2_any.at[pe, :, pl.ds(pc * FH, FH)],
                              w2s.at[slot], sem2.at[slot]).start()

    def wait_pair_chunk(pe, pc, slot):
        pltpu.make_async_copy(w1_any.at[pe, pl.ds(pc * FH, FH), :],
                              w1s.at[slot], sem1.at[slot]).wait()
        pltpu.make_async_copy(w2_any.at[pe, :, pl.ds(pc * FH, FH)],
                              w2s.at[slot], sem2.at[slot]).wait()

    @pl.when((t == 0) & (fc == 0))
    def _():
        start_pair_chunk(e, 0, 0)
        start_pair_chunk(e, 1, 1)

    @pl.when(fc == 1)
    def _():
        pa = tab_ref[TB_PA, t]

        @pl.when(pa >= 0)
        def _():
            start_pair_chunk(pa // 4, 0, lax.rem(pa, 4))

        pb = tab_ref[TB_PB, t]

        @pl.when(pb >= 0)
        def _():
            start_pair_chunk(pb // 4, 1, lax.rem(pb, 4))

    @pl.when((tab_ref[TB_Z, t] == 1) & (fc == 0))
    def _():
        y_ref[...] = jnp.zeros_like(y_ref)

    @pl.when(tab_ref[TB_V, t] == 1)
    def _():
        slot = jnp.where(fc == 0, tab_ref[TB_CS0, t], tab_ref[TB_CS1, t])

        @pl.when(tab_ref[TB_WF, t] == 1)
        def _():
            wait_pair_chunk(e, fc, slot)

        lo = soff_ref[e]
        hi = soff_ref[e + 1]
        xb = x_ref[...]
        h = lax.dot_general(xb, w1s[slot], (((1,), (1,)), ((), ())),
                            preferred_element_type=jnp.float32)
        h = 0.5 * h * (1.0 + lax.erf(h * 0.7071067811865476))
        y = lax.dot_general(h, w2s[slot], (((1,), (1,)), ((), ())),
                            preferred_element_type=jnp.float32)
        rows = b * BLK + lax.broadcasted_iota(jnp.int32, (BLK, 1), 0)
        mask = (rows >= lo) & (rows < hi)
        y_ref[...] += jnp.where(mask, y, 0.0)


def _ffn_call(tab, offsets, xs, W1, W2):
    grid_spec = pltpu.PrefetchScalarGridSpec(
        num_scalar_prefetch=2,
        grid=(T, FC),
        in_specs=[
            pl.BlockSpec((BLK, D), lambda t, fc, tab, so: (tab[TB_B, t], 0)),
            pl.BlockSpec(memory_space=pl.ANY),
            pl.BlockSpec(memory_space=pl.ANY),
        ],
        out_specs=pl.BlockSpec((BLK, D),
                               lambda t, fc, tab, so: (tab[TB_B, t], 0)),
        scratch_shapes=[
            pltpu.VMEM((NSLOT, FH, D), jnp.float32),
            pltpu.VMEM((NSLOT, D, FH), jnp.float32),
            pltpu.SemaphoreType.DMA((NSLOT,)),
            pltpu.SemaphoreType.DMA((NSLOT,)),
        ],
    )
    return pl.pallas_call(
        _ffn_body,
        grid_spec=grid_spec,
        out_shape=jax.ShapeDtypeStruct((A, D), jnp.float32),
    )(tab, offsets, xs, W1, W2)


def _combine_body(x_ref, meta_ref, y0_ref, y1_ref, o_ref):
    g1 = meta_ref[0, :, 4:5]
    g2 = meta_ref[0, :, 5:6]
    o_ref[...] = x_ref[...] + g1 * y0_ref[...] + g2 * y1_ref[...]


def _combine_call(xm, meta, yu):
    return pl.pallas_call(
        _combine_body,
        grid=(NBR,),
        in_specs=[
            pl.BlockSpec((BT, D), lambda i: (i, 0)),
            pl.BlockSpec((1, BT, E), lambda i: (i, 0, 0)),
            pl.BlockSpec((BT, D), lambda i: (i, 0)),
            pl.BlockSpec((BT, D), lambda i: (i + NBR, 0)),
        ],
        out_specs=pl.BlockSpec((BT, D), lambda i: (i, 0)),
        out_shape=jax.ShapeDtypeStruct((S, D), jnp.float32),
    )(xm, meta, yu, yu)


def kernel(x, gate_W, W1, W2, ln_gamma, ln_beta):
    xm = x.reshape(S, D)
    gamma = ln_gamma.reshape(1, D)
    beta = ln_beta.reshape(1, D)

    xn, meta, counts_f, aux = _router_call(xm, gate_W, gamma, beta)
    counts = counts_f[0].astype(jnp.int32)

    e_flat = jnp.concatenate(
        [meta[:, :, 0].reshape(S), meta[:, :, 1].reshape(S)]).astype(jnp.int32)
    r_flat = jnp.concatenate(
        [meta[:, :, 2].reshape(S), meta[:, :, 3].reshape(S)]).astype(jnp.int32)
    offsets = jnp.concatenate(
        [jnp.zeros((1,), jnp.int32), jnp.cumsum(counts)]).astype(jnp.int32)
    off16 = jnp.concatenate([offsets[:E], jnp.zeros((8,), jnp.int32)])

    pos, stok = _dispatch_call(e_flat, r_flat, off16)

    xs = _gather_rows_call(stok.reshape(A // 16, 16), xn)

    lo_blk = offsets[:E] // BLK
    hi_blk = jnp.maximum(offsets[1:] - 1, 0) // BLK
    nblk = jnp.where(counts > 0, hi_blk - lo_blk + 1, 0)
    cum = jnp.cumsum(nblk)
    starts = cum - nblk
    total = cum[E - 1]
    t = jnp.arange(T)
    te = jnp.minimum(t, total - 1)
    e_t = jnp.searchsorted(cum, te, side="right").astype(jnp.int32)
    b_t = (lo_blk[e_t] + (te - starts[e_t])).astype(jnp.int32)
    valid = t < total
    prevb = jnp.concatenate([jnp.full((1,), -1, jnp.int32), b_t[:-1]])
    first_blk = (b_t != prevb) & valid
    preve = jnp.concatenate([jnp.full((1,), -1, jnp.int32), e_t[:-1]])
    run_first = valid & ((t == 0) | (e_t != preve))
    nexte = jnp.concatenate([e_t[1:], jnp.full((1,), -1, jnp.int32)])
    nextv = jnp.concatenate([valid[1:], jnp.zeros((1,), bool)])
    run_last = valid & (~nextv | (nexte != e_t))
    runidx = jnp.cumsum(run_first.astype(jnp.int32)) - 1
    cs0 = (2 * runidx) % NSLOT
    cs1 = (2 * runidx + 1) % NSLOT
    idxs = jnp.where(run_first, t, T)
    sh = jnp.concatenate([idxs[1:], jnp.full((1,), T, jnp.int32)])
    nf = jnp.flip(lax.cummin(jnp.flip(sh)))
    has_next = nf < T
    ne = e_t[jnp.clip(nf, 0, T - 1)]
    pa = jnp.where(run_first & has_next,
                   ne * 4 + (2 * (runidx + 1)) % NSLOT, -1)
    pb = jnp.where(run_last & has_next,
                   ne * 4 + (2 * (runidx + 1) + 1) % NSLOT, -1)
    tab = jnp.stack([e_t, b_t, first_blk.astype(jnp.int32),
                     valid.astype(jnp.int32), run_first.astype(jnp.int32),
                     cs0, cs1, pa, pb]).astype(jnp.int32)

    ys = _ffn_call(tab, offsets, xs, W1, W2)

    yu = _gather_rows_call(pos.reshape(A // 16, 16), ys)

    out = _combine_call(xm, meta, yu)
    return out.reshape(1, S, D), aux[0, 0]


# speedup vs baseline: 1.4272x; 1.4272x over previous
import functools

import jax
import jax.numpy as jnp
from jax import lax
from jax.experimental import pallas as pl
from jax.experimental.pallas import tpu as pltpu
from jax.experimental.pallas import tpu_sc as plsc

D = 2048
E = 8
F = 2048
S = 2048
K = 2
A = S * K
LBW = 0.01

BT = 256
NBR = S // BT
BLK = 256
NB = A // BLK
T = NB + E - 1


def _router_body(x_ref, gw_ref, gamma_ref, beta_ref,
                 xn_ref, meta_ref, counts_ref, aux_ref, run_ref, psum_ref):
    i = pl.program_id(0)

    @pl.when(i == 0)
    def _():
        run_ref[...] = jnp.zeros((1, E), jnp.float32)
        psum_ref[...] = jnp.zeros((1, E), jnp.float32)

    xb = x_ref[...]
    mean = jnp.mean(xb, axis=1, keepdims=True)
    xc = xb - mean
    var = jnp.mean(xc * xc, axis=1, keepdims=True)
    xn = xc / jnp.sqrt(var + 1e-5) * gamma_ref[...] + beta_ref[...]
    xn_ref[...] = xn

    logits = lax.dot_general(xn.astype(jnp.bfloat16),
                             gw_ref[...].astype(jnp.bfloat16),
                             (((1,), (1,)), ((), ())),
                             preferred_element_type=jnp.float32)

    ecol = lax.broadcasted_iota(jnp.int32, (BT, E), 1)
    m1 = jnp.max(logits, axis=1, keepdims=True)
    i1 = jnp.min(jnp.where(logits == m1, ecol, E), axis=1, keepdims=True)
    l2 = jnp.where(ecol == i1, jnp.float32(-1e30), logits)
    m2 = jnp.max(l2, axis=1, keepdims=True)
    i2 = jnp.min(jnp.where(l2 == m2, ecol, E), axis=1, keepdims=True)

    ed = jnp.exp(m2 - m1)
    g1 = 1.0 / (1.0 + ed)
    g2 = ed / (1.0 + ed)

    p = jnp.exp(logits - m1)
    p = p / jnp.sum(p, axis=1, keepdims=True)
    psum_ref[...] += jnp.sum(p, axis=0, keepdims=True)

    eh = jnp.concatenate([i1, i2], axis=0)
    onehot = (eh == lax.broadcasted_iota(jnp.int32, (2 * BT, E), 1)
              ).astype(jnp.float32)
    rl = lax.broadcasted_iota(jnp.int32, (2 * BT, 2 * BT), 0)
    cl = lax.broadcasted_iota(jnp.int32, (2 * BT, 2 * BT), 1)
    tril = (cl < rl).astype(jnp.float32)
    csum = lax.dot_general(tril, onehot, (((1,), (0,)), ((), ())),
                           preferred_element_type=jnp.float32)
    run = run_ref[...]
    rank = jnp.sum((csum + run) * onehot, axis=1, keepdims=True)
    run_ref[...] = run + jnp.sum(onehot, axis=0, keepdims=True)
    counts_ref[...] = run_ref[...]

    z = jnp.zeros((BT, 1), jnp.float32)
    meta_ref[...] = jnp.concatenate(
        [i1.astype(jnp.float32), i2.astype(jnp.float32),
         rank[:BT], rank[BT:], g1, g2, z, z], axis=1)[None]

    @pl.when(i == NBR - 1)
    def _():
        avg = psum_ref[...] / S
        mu = jnp.mean(avg)
        varu = jnp.sum((avg - mu) ** 2) / (E - 1)
        aux_ref[...] = jnp.full((1, E), varu / ((mu + 1e-6) ** 2) * LBW,
                                jnp.float32)


def _router_call(xm, gate_W, gamma, beta):
    return pl.pallas_call(
        _router_body,
        grid=(NBR,),
        in_specs=[
            pl.BlockSpec((BT, D), lambda i: (i, 0)),
            pl.BlockSpec((E, D), lambda i: (0, 0)),
            pl.BlockSpec((1, D), lambda i: (0, 0)),
            pl.BlockSpec((1, D), lambda i: (0, 0)),
        ],
        out_specs=[
            pl.BlockSpec((BT, D), lambda i: (i, 0)),
            pl.BlockSpec((1, BT, E), lambda i: (i, 0, 0)),
            pl.BlockSpec((1, E), lambda i: (0, 0)),
            pl.BlockSpec((1, E), lambda i: (0, 0)),
        ],
        out_shape=[
            jax.ShapeDtypeStruct((S, D), jnp.float32),
            jax.ShapeDtypeStruct((NBR, BT, E), jnp.float32),
            jax.ShapeDtypeStruct((1, E), jnp.float32),
            jax.ShapeDtypeStruct((1, E), jnp.float32),
        ],
        scratch_shapes=[
            pltpu.VMEM((1, E), jnp.float32),
            pltpu.VMEM((1, E), jnp.float32),
        ],
    )(xm, gate_W, gamma, beta)


def _dispatch_call(e_flat, r_flat, off16):
    mesh = plsc.VectorSubcoreMesh(core_axis_name="c", subcore_axis_name="s")

    @functools.partial(
        pl.kernel,
        out_type=(jax.ShapeDtypeStruct((A,), jnp.int32),
                  jax.ShapeDtypeStruct((A,), jnp.int32)),
        mesh=mesh,
        compiler_params=pltpu.CompilerParams(needs_layout_passes=False),
        scratch_types=[
            pltpu.VMEM((16,), jnp.int32),
            pltpu.VMEM((A,), jnp.int32),
            pltpu.VMEM((A,), jnp.int32),
            pltpu.VMEM((A,), jnp.int32),
            pltpu.VMEM((A,), jnp.int32),
        ],
    )
    def k(e_hbm, r_hbm, c_hbm, pos_hbm, stok_hbm, off_v, e_v, r_v, pos_v,
          stok_v):
        c = lax.axis_index("c")
        s = lax.axis_index("s")

        @pl.when((c == 0) & (s == 0))
        def _():
            pltpu.sync_copy(c_hbm, off_v)
            pltpu.sync_copy(e_hbm, e_v)
            pltpu.sync_copy(r_hbm, r_v)

            def body(i, carry):
                ev = e_v[pl.ds(i * 16, 16)]
                rv = r_v[pl.ds(i * 16, 16)]
                ps = rv + plsc.load_gather(off_v, [ev])
                pos_v[pl.ds(i * 16, 16)] = ps
                a = lax.iota(jnp.int32, 16) + i * 16
                plsc.store_scatter(stok_v, [ps], lax.rem(a, S))
                return carry

            lax.fori_loop(0, A // 16, body, 0)
            pltpu.sync_copy(pos_v, pos_hbm)
            pltpu.sync_copy(stok_v, stok_hbm)

    return k(e_flat, r_flat, off16)


def _gather_rows_call(idx2, src):
    mesh = plsc.VectorSubcoreMesh(core_axis_name="c", subcore_axis_name="s")
    dt = src.dtype
    D2 = src.shape[1]

    @functools.partial(
        pl.kernel,
        out_type=jax.ShapeDtypeStruct((A, D2), dt),
        mesh=mesh,
        compiler_params=pltpu.CompilerParams(needs_layout_passes=False),
        scratch_types=[
            pltpu.VMEM((8, 16), jnp.int32),
            pltpu.VMEM((16, D2), dt),
            pltpu.VMEM((16, D2), dt),
            pltpu.SemaphoreType.DMA,
            pltpu.SemaphoreType.DMA,
        ],
    )
    def k(idx_hbm, src_hbm, out_hbm, idx_v, buf0, buf1, sem0, sem1):
        c = lax.axis_index("c")
        s = lax.axis_index("s")
        wid = s * 2 + c
        pltpu.sync_copy(idx_hbm.at[pl.ds(wid * 8, 8)], idx_v)
        bufs = (buf0, buf1)
        sems = (sem0, sem1)
        pend = {0: pltpu.async_copy(src_hbm.at[idx_v.at[0]], buf0, sem0)}
        for j in range(8):
            if j + 1 < 8:
                pend[(j + 1) % 2] = pltpu.async_copy(
                    src_hbm.at[idx_v.at[j + 1]], bufs[(j + 1) % 2],
                    sems[(j + 1) % 2])
            pend[j % 2].wait()
            pltpu.sync_copy(bufs[j % 2],
                            out_hbm.at[pl.ds(wid * 128 + j * 16, 16)])

    return k(idx2, src)


FC = 2
FH = F // FC
NSLOT = 3

TB_E, TB_B, TB_Z, TB_V, TB_WF, TB_CS0, TB_CS1, TB_PA, TB_PB = range(9)


def _ffn_body(tab_ref, soff_ref, x_ref, w1_any, w2_any, y_ref,
              w1s, w2s, sem1, sem2):
    t = pl.program_id(0)
    fc = pl.program_id(1)
    e = tab_ref[TB_E, t]
    b = tab_ref[TB_B, t]

    def start_pair_chunk(pe, pc, slot):
        pltpu.make_async_copy(w1_any.at[pe, pl.ds(pc * FH, FH), :],
                              w1s.at[slot], sem1.at[slot]).start()
        pltpu.make_async_copy(w2_any.at[pe, :, pl.ds(pc * FH, FH)],
                              w2s.at[slot], sem2.at[slot]).start()

    def wait_pair_chunk(pe, pc, slot):
        pltpu.make_async_copy(w1_any.at[pe, pl.ds(pc * FH, FH), :],
                              w1s.at[slot], sem1.at[slot]).wait()
        pltpu.make_async_copy(w2_any.at[pe, :, pl.ds(pc * FH, FH)],
                              w2s.at[slot], sem2.at[slot]).wait()

    @pl.when((t == 0) & (fc == 0))
    def _():
        start_pair_chunk(e, 0, 0)
        start_pair_chunk(e, 1, 1)

    @pl.when(fc == 1)
    def _():
        pa = tab_ref[TB_PA, t]

        @pl.when(pa >= 0)
        def _():
            start_pair_chunk(pa // 4, 0, lax.rem(pa, 4))

        pb = tab_ref[TB_PB, t]

        @pl.when(pb >= 0)
        def _():
            start_pair_chunk(pb // 4, 1, lax.rem(pb, 4))

    @pl.when((tab_ref[TB_Z, t] == 1) & (fc == 0))
    def _():
        y_ref[...] = jnp.zeros_like(y_ref)

    @pl.when(tab_ref[TB_V, t] == 1)
    def _():
        slot = jnp.where(fc == 0, tab_ref[TB_CS0, t], tab_ref[TB_CS1, t])

        @pl.when(tab_ref[TB_WF, t] == 1)
        def _():
            wait_pair_chunk(e, fc, slot)

        lo = soff_ref[e]
        hi = soff_ref[e + 1]
        xb = x_ref[...]
        h = lax.dot_general(xb, w1s[slot], (((1,), (1,)), ((), ())),
                            preferred_element_type=jnp.float32)
        h = 0.5 * h * (1.0 + lax.erf(h * 0.7071067811865476))
        y = lax.dot_general(h, w2s[slot], (((1,), (1,)), ((), ())),
                            preferred_element_type=jnp.float32)
        rows = b * BLK + lax.broadcasted_iota(jnp.int32, (BLK, 1), 0)
        mask = (rows >= lo) & (rows < hi)
        y_ref[...] += jnp.where(mask, y, 0.0)


def _ffn_call(tab, offsets, xs, W1, W2):
    grid_spec = pltpu.PrefetchScalarGridSpec(
        num_scalar_prefetch=2,
        grid=(T, FC),
        in_specs=[
            pl.BlockSpec((BLK, D), lambda t, fc, tab, so: (tab[TB_B, t], 0)),
            pl.BlockSpec(memory_space=pl.ANY),
            pl.BlockSpec(memory_space=pl.ANY),
        ],
        out_specs=pl.BlockSpec((BLK, D),
                               lambda t, fc, tab, so: (tab[TB_B, t], 0)),
        scratch_shapes=[
            pltpu.VMEM((NSLOT, FH, D), jnp.float32),
            pltpu.VMEM((NSLOT, D, FH), jnp.float32),
            pltpu.SemaphoreType.DMA((NSLOT,)),
            pltpu.SemaphoreType.DMA((NSLOT,)),
        ],
    )
    return pl.pallas_call(
        _ffn_body,
        grid_spec=grid_spec,
        out_shape=jax.ShapeDtypeStruct((A, D), jnp.float32),
    )(tab, offsets, xs, W1, W2)


def _combine_body(x_ref, meta_ref, y0_ref, y1_ref, o_ref):
    g1 = meta_ref[0, :, 4:5]
    g2 = meta_ref[0, :, 5:6]
    o_ref[...] = x_ref[...] + g1 * y0_ref[...] + g2 * y1_ref[...]


def _combine_call(xm, meta, yu):
    return pl.pallas_call(
        _combine_body,
        grid=(NBR,),
        in_specs=[
            pl.BlockSpec((BT, D), lambda i: (i, 0)),
            pl.BlockSpec((1, BT, E), lambda i: (i, 0, 0)),
            pl.BlockSpec((BT, D), lambda i: (i, 0)),
            pl.BlockSpec((BT, D), lambda i: (i + NBR, 0)),
        ],
        out_specs=pl.BlockSpec((BT, D), lambda i: (i, 0)),
        out_shape=jax.ShapeDtypeStruct((S, D), jnp.float32),
    )(xm, meta, yu, yu)


def kernel(x, gate_W, W1, W2, ln_gamma, ln_beta):
    xm = x.reshape(S, D)
    gamma = ln_gamma.reshape(1, D)
    beta = ln_beta.reshape(1, D)

    xn, meta, counts_f, aux = _router_call(xm, gate_W, gamma, beta)
    counts = counts_f[0].astype(jnp.int32)

    e_flat = jnp.concatenate(
        [meta[:, :, 0].reshape(S), meta[:, :, 1].reshape(S)]).astype(jnp.int32)
    r_flat = jnp.concatenate(
        [meta[:, :, 2].reshape(S), meta[:, :, 3].reshape(S)]).astype(jnp.int32)
    offsets = jnp.concatenate(
        [jnp.zeros((1,), jnp.int32), jnp.cumsum(counts)]).astype(jnp.int32)
    off16 = jnp.concatenate([offsets[:E], jnp.zeros((8,), jnp.int32)])

    pos, stok = _dispatch_call(e_flat, r_flat, off16)

    xs = _gather_rows_call(stok.reshape(A // 16, 16), xn)

    lo_blk = offsets[:E] // BLK
    hi_blk = jnp.maximum(offsets[1:] - 1, 0) // BLK
    nblk = jnp.where(counts > 0, hi_blk - lo_blk + 1, 0)
    cum = jnp.cumsum(nblk)
    starts = cum - nblk
    total = cum[E - 1]
    t = jnp.arange(T)
    te = jnp.minimum(t, total - 1)
    e_t = jnp.searchsorted(cum, te, side="right").astype(jnp.int32)
    b_t = (lo_blk[e_t] + (te - starts[e_t])).astype(jnp.int32)
    valid = t < total
    prevb = jnp.concatenate([jnp.full((1,), -1, jnp.int32), b_t[:-1]])
    first_blk = (b_t != prevb) & valid
    preve = jnp.concatenate([jnp.full((1,), -1, jnp.int32), e_t[:-1]])
    run_first = valid & ((t == 0) | (e_t != preve))
    nexte = jnp.concatenate([e_t[1:], jnp.full((1,), -1, jnp.int32)])
    nextv = jnp.concatenate([valid[1:], jnp.zeros((1,), bool)])
    run_last = valid & (~nextv | (nexte != e_t))
    runidx = jnp.cumsum(run_first.astype(jnp.int32)) - 1
    cs0 = (2 * runidx) % NSLOT
    cs1 = (2 * runidx + 1) % NSLOT
    idxs = jnp.where(run_first, t, T)
    sh = jnp.concatenate([idxs[1:], jnp.full((1,), T, jnp.int32)])
    nf = jnp.flip(lax.cummin(jnp.flip(sh)))
    has_next = nf < T
    ne = e_t[jnp.clip(nf, 0, T - 1)]
    pa = jnp.where(run_first & has_next,
                   ne * 4 + (2 * (runidx + 1)) % NSLOT, -1)
    pb = jnp.where(run_last & has_next,
                   ne * 4 + (2 * (runidx + 1) + 1) % NSLOT, -1)
    tab = jnp.stack([e_t, b_t, first_blk.astype(jnp.int32),
                     valid.astype(jnp.int32), run_first.astype(jnp.int32),
                     cs0, cs1, pa, pb]).astype(jnp.int32)

    ys = _ffn_call(tab, offsets, xs, W1, W2)

    yu = _gather_rows_call(pos.reshape(A // 16, 16), ys)

    out = _combine_call(xm, meta, yu)
    return out.reshape(1, S, D), aux[0, 0]
